# Initial kernel scaffold; baseline (speedup 1.0000x reference)
import jax
import jax.numpy as jnp
from jax.experimental import pallas as pl


def kernel(x_sentance, x_token, x_token_ori, edge_index, W_proj, b_proj, W_gnn, b_gnn):
    raise NotImplementedError("write your pallas kernel here")


# speedup vs baseline: 12.7312x; 12.7312x over previous
import functools

import jax
import jax.numpy as jnp
from jax import lax
from jax.experimental import pallas as pl
from jax.experimental.pallas import tpu as pltpu
from jax.experimental.pallas import tpu_sc as plsc

NC = 2
NS = 16
LANES = 16
D = 128
W = 128


@functools.cache
def _mesh():
    return plsc.VectorSubcoreMesh(
        core_axis_name="c", subcore_axis_name="s",
        num_cores=NC, num_subcores=NS)


def _loop32(n, step, body):
    def wrapped(i, carry):
        body(i * jnp.int32(step))
        return carry
    lax.fori_loop(jnp.int32(0), jnp.int32(n // step), wrapped, None)


def _zero_fill_1d(ref, n):
    def bod(i):
        ref[pl.ds(i, LANES)] = jnp.zeros((LANES,), jnp.float32)
    _loop32(n, LANES, bod)


def _zero_fill_2d(ref, rows):
    def outer(r):
        def inner(i):
            ref[r, pl.ds(i, LANES)] = jnp.zeros((LANES,), jnp.float32)
        _loop32(D, LANES, inner)
    _loop32(rows, 1, outer)


def _make_hist(npad, epad):
    per_tile = epad // (NC * NS)
    rows_per_tile = npad // NS

    @functools.partial(
        pl.kernel,
        out_type=jax.ShapeDtypeStruct((NC * npad,), jnp.float32),
        mesh=_mesh(),
        scratch_types=[
            pltpu.VMEM_SHARED((npad,), jnp.float32),
            pltpu.VMEM((W,), jnp.int32),
            pltpu.VMEM((W,), jnp.float32),
            pltpu.VMEM((rows_per_tile,), jnp.float32),
        ],
    )
    def hist_kernel(dst_hbm, h_hbm, hist_sp, idx_v, ones_v, zb_v):
        c = lax.axis_index("c")
        s = lax.axis_index("s")
        wid = c * jnp.int32(NS) + s

        def fill_ones(i):
            ones_v[pl.ds(i, LANES)] = jnp.ones((LANES,), jnp.float32)
        _loop32(W, LANES, fill_ones)
        _zero_fill_1d(zb_v, rows_per_tile)
        r0 = s * jnp.int32(rows_per_tile)
        pltpu.sync_copy(zb_v, hist_sp.at[pl.ds(r0, rows_per_tile)])
        plsc.subcore_barrier()

        base = wid * jnp.int32(per_tile)

        def hist_win(j):
            pltpu.sync_copy(dst_hbm.at[pl.ds(base + j, W)], idx_v)
            pltpu.sync_copy(ones_v, hist_sp.at[idx_v], add=True)
        _loop32(per_tile, W, hist_win)

        plsc.subcore_barrier()
        pltpu.sync_copy(hist_sp.at[pl.ds(r0, rows_per_tile)],
                        h_hbm.at[pl.ds(c * jnp.int32(npad) + r0, rows_per_tile)])

    return hist_kernel


def _make_scatter(npad, epad):
    per_tile = epad // NS
    rows_per_tile = npad // NS

    @functools.partial(
        pl.kernel,
        out_type=jax.ShapeDtypeStruct((NC * npad, D), jnp.float32),
        mesh=_mesh(),
        scratch_types=[
            pltpu.VMEM_SHARED((npad, D), jnp.float32),
            pltpu.VMEM((W,), jnp.int32),
            pltpu.VMEM((W,), jnp.int32),
            pltpu.VMEM((W, D), jnp.float32),
            pltpu.VMEM((W, D), jnp.float32),
            pltpu.SemaphoreType.DMA,
        ],
    )
    def scatter_kernel(y_hbm, src_hbm, dst_hbm, s_hbm,
                       acc_sp, sidx_v, didx_v, rows_v, zb_v, sem):
        c = lax.axis_index("c")
        s = lax.axis_index("s")
        tile_r0 = s * jnp.int32(rows_per_tile)

        _zero_fill_2d(zb_v, W)

        def zero_chunk(j):
            pltpu.sync_copy(zb_v, acc_sp.at[pl.ds(tile_r0 + j, W), :])
        _loop32(rows_per_tile, W, zero_chunk)
        plsc.subcore_barrier()

        ebase = c * jnp.int32(epad) + s * jnp.int32(per_tile)
        dbase = s * jnp.int32(per_tile)

        def win(j):
            pltpu.sync_copy(src_hbm.at[pl.ds(ebase + j, W)], sidx_v)
            pltpu.sync_copy(dst_hbm.at[pl.ds(dbase + j, W)], didx_v)
            pltpu.async_copy(y_hbm.at[sidx_v], rows_v, sem).wait()
            pltpu.sync_copy(rows_v, acc_sp.at[didx_v], add=True)
        _loop32(per_tile, W, win)

        plsc.subcore_barrier()

        out_r0 = c * jnp.int32(npad) + tile_r0

        def wchunk(j):
            pltpu.sync_copy(acc_sp.at[pl.ds(tile_r0 + j, W), :],
                            s_hbm.at[pl.ds(out_r0 + j, W), :])
        _loop32(rows_per_tile, W, wchunk)

    return scatter_kernel


_DOT_KW = dict(preferred_element_type=jnp.float32,
               precision=lax.Precision.HIGHEST)


def _prep_body(xs_ref, xt_ref, h0_ref, h1_ref, wp_ref, wg_ref,
               y_ref, d_ref):
    deg = h0_ref[...] + h1_ref[...] + 1.0
    dval = lax.rsqrt(deg)
    d_ref[...] = dval
    xs = xs_ref[...]
    y1 = lax.dot_general(xs, wp_ref[...], (((1,), (1,)), ((), ())), **_DOT_KW)
    y_ref[0] = y1 * dval
    y2 = lax.dot_general(xs + xt_ref[...], wg_ref[...],
                         (((1,), (1,)), ((), ())), **_DOT_KW)
    y_ref[1] = y2 * dval


def _make_prep(npad, rb):
    grid = npad // rb
    row_spec = pl.BlockSpec((rb, D), lambda i: (i, 0))
    col_spec = pl.BlockSpec((rb, 1), lambda i: (i, 0))
    w_spec = pl.BlockSpec((D, D), lambda i: (0, 0))
    return pl.pallas_call(
        _prep_body,
        grid=(grid,),
        in_specs=[row_spec, row_spec, col_spec, col_spec, w_spec, w_spec],
        out_specs=[pl.BlockSpec((2, rb, D), lambda i: (0, i, 0)), col_spec],
        out_shape=(jax.ShapeDtypeStruct((2, npad, D), jnp.float32),
                   jax.ShapeDtypeStruct((npad, 1), jnp.float32)),
    )


def _make_final(n, npad, rb):
    grid = n // rb
    nrows_f = float(n)
    off = npad // rb

    def body(s1_ref, y1_ref, s2_ref, y2_ref, d_ref, xs_ref, xo_ref,
             bp_ref, bg_ref, es_ref, et_ref, cs_ref, ct_ref, accs, acct):
        i = pl.program_id(0)
        dval = d_ref[...]
        xs = xs_ref[...]
        es = dval * (s1_ref[...] + y1_ref[...]) + bp_ref[...] + xs
        et = xo_ref[...] + dval * (s2_ref[...] + y2_ref[...]) + bg_ref[...] + xs
        es_ref[...] = es
        et_ref[...] = et
        ps = jnp.sum(es, axis=0, keepdims=True)
        pt = jnp.sum(et)

        @pl.when(i == 0)
        def _():
            accs[...] = jnp.zeros_like(accs)
            acct[0] = 0.0

        accs[0:1, :] += ps
        acct[0] += pt

        @pl.when(i == grid - 1)
        def _():
            cs_ref[...] = accs[0:1, :] * (1.0 / nrows_f)
            ct_ref[...] = jnp.full((1, 1), acct[0] * (1.0 / (nrows_f * D)),
                                   jnp.float32)

    lo_spec = pl.BlockSpec((rb, D), lambda i: (i, 0))
    hi_spec = pl.BlockSpec((rb, D), lambda i: (i + off, 0))
    col_spec = pl.BlockSpec((rb, 1), lambda i: (i, 0))
    b_spec = pl.BlockSpec((1, D), lambda i: (0, 0))
    return pl.pallas_call(
        body,
        grid=(grid,),
        in_specs=[lo_spec, lo_spec, hi_spec, hi_spec, col_spec,
                  lo_spec, lo_spec, b_spec, b_spec],
        out_specs=[lo_spec, lo_spec,
                   pl.BlockSpec((1, D), lambda i: (0, 0)),
                   pl.BlockSpec((1, 1), lambda i: (0, 0))],
        out_shape=(jax.ShapeDtypeStruct((n, D), jnp.float32),
                   jax.ShapeDtypeStruct((n, D), jnp.float32),
                   jax.ShapeDtypeStruct((1, D), jnp.float32),
                   jax.ShapeDtypeStruct((1, 1), jnp.float32)),
        scratch_shapes=[pltpu.VMEM((8, D), jnp.float32),
                        pltpu.SMEM((1,), jnp.float32)],
    )


def kernel(x_sentance, x_token, x_token_ori, edge_index,
           W_proj, b_proj, W_gnn, b_gnn):
    with jax.enable_x64(False):
        return _kernel_impl(x_sentance, x_token, x_token_ori, edge_index,
                            W_proj, b_proj, W_gnn, b_gnn)


def _kernel_impl(x_sentance, x_token, x_token_ori, edge_index,
                 W_proj, b_proj, W_gnn, b_gnn):
    n, d = x_sentance.shape
    e = edge_index.shape[1]
    assert d == D
    npad = -(-n // 2048) * 2048
    ewin = NC * NS * W
    epad = -(-e // ewin) * ewin

    src = edge_index[0].astype(jnp.int32)
    dst = edge_index[1].astype(jnp.int32)
    npadrows = npad - n
    pad_idx = n + (jnp.arange(epad - e, dtype=jnp.int32) % npadrows)
    src_p = jnp.concatenate([src, pad_idx])
    dst_p = jnp.concatenate([dst, pad_idx])
    src_cc = jnp.concatenate([src_p, src_p + jnp.int32(npad)])
    zrows = jnp.zeros((npadrows, D), jnp.float32)
    xs_p = jnp.concatenate([x_sentance, zrows])
    xt_p = jnp.concatenate([x_token, zrows])

    h = _make_hist(npad, epad)(dst_p)
    hcol = h.reshape(NC * npad, 1)
    ycat, dcol = _make_prep(npad, 80)(
        xs_p, xt_p, hcol[:npad], hcol[npad:], W_proj, W_gnn)
    y_flat = ycat.reshape(NC * npad, D)
    s_flat = _make_scatter(npad, epad)(y_flat, src_cc, dst_p)
    emb_s, emb_t, ctx_s, ctx_t = _make_final(n, npad, 80)(
        s_flat, y_flat, s_flat, y_flat, dcol, x_sentance, x_token_ori,
        b_proj.reshape(1, D), b_gnn.reshape(1, D))
    return emb_s, emb_t, ctx_s.reshape(D), ctx_t.reshape(())
